# baseline (speedup 1.0000x reference)
import functools

import jax
import jax.numpy as jnp
from jax import lax
from jax.experimental import pallas as pl
from jax.experimental.pallas import tpu as pltpu
from jax.experimental.pallas import tpu_sc as plsc

NC = 2
NS = 16
NW = NC * NS
L = 16


@functools.partial(jax.jit, static_argnums=(1,))
def _unique_counts_neg(neighbors_flat, n):
    rows_per_w = n // NW
    blk = 8
    nblk = rows_per_w // blk
    chunks = n // L

    mesh = plsc.VectorSubcoreMesh(core_axis_name="c", subcore_axis_name="s")

    @functools.partial(
        pl.kernel,
        out_type=jax.ShapeDtypeStruct((n,), jnp.float32),
        mesh=mesh,
        compiler_params=pltpu.CompilerParams(needs_layout_passes=False),
        scratch_types=[
            pltpu.VMEM((blk * n,), jnp.int32),
            pltpu.VMEM((blk * n,), jnp.int32),
            pltpu.VMEM((n,), jnp.int32),
            pltpu.VMEM((rows_per_w,), jnp.float32),
            pltpu.SemaphoreType.DMA,
            pltpu.SemaphoreType.DMA,
        ],
    )
    def body(nb_hbm, out_hbm, buf0, buf1, mark, res, sem0, sem1):
        cid = lax.axis_index("c")
        sid = lax.axis_index("s")
        wid = sid * NC + cid
        base_row = wid * rows_per_w
        bufs = (buf0, buf1)
        sems = (sem0, sem1)

        lanes = lax.iota(jnp.int32, L)
        mask0 = lanes == 0
        neg1 = jnp.full((L,), -1, jnp.int32)

        def start_block(b, par):
            src = nb_hbm.at[pl.ds((base_row + b * blk) * n, blk * n)]
            pltpu.make_async_copy(src, bufs[par], sems[par]).start()

        def wait_block(par):
            src = nb_hbm.at[pl.ds(0, blk * n)]
            pltpu.make_async_copy(src, bufs[par], sems[par]).wait()

        @plsc.parallel_loop(0, chunks, unroll=16)
        def _(i):
            mark[pl.ds(i * L, L)] = neg1

        start_block(0, 0)
        start_block(1, 1)

        def pair_body(bp, carry):
            for par in range(2):
                b = 2 * bp + par
                wait_block(par)
                rowbuf = bufs[par]

                for r in range(blk):
                    k = b * blk + r
                    kvec = jnp.full((L,), k, jnp.int32)
                    rbase = r * n

                    @plsc.parallel_loop(0, chunks, unroll=16)
                    def _(j):
                        idx = rowbuf[pl.ds(rbase + j * L, L)]
                        plsc.store_scatter(mark, [idx], kvec)

                    @plsc.parallel_loop(
                        0, chunks, unroll=16, carry=jnp.zeros((L,), jnp.int32)
                    )
                    def acc(j, a):
                        m = mark[pl.ds(j * L, L)]
                        return a + (m == kvec).astype(jnp.int32)

                    total = jnp.sum(acc)
                    val = jnp.full((L,), 0.0, jnp.float32) - total.astype(
                        jnp.float32
                    )
                    plsc.store_scatter(
                        res, [jnp.full((L,), k, jnp.int32)], val, mask=mask0
                    )

                @pl.when(b + 2 < nblk)
                def _():
                    start_block(b + 2, par)
            return carry

        lax.fori_loop(0, nblk // 2, pair_body, 0)
        pltpu.sync_copy(res, out_hbm.at[pl.ds(base_row, rows_per_w)])

    return body(neighbors_flat)


def kernel(vision_features, neighbors, gt, num_views):
    n = neighbors.shape[0]
    return _unique_counts_neg(neighbors.reshape(-1), n)


# speedup vs baseline: 1.0158x; 1.0158x over previous
import functools

import jax
import jax.numpy as jnp
from jax import lax
from jax.experimental import pallas as pl
from jax.experimental.pallas import tpu as pltpu
from jax.experimental.pallas import tpu_sc as plsc

NC = 2
NS = 16
NW = NC * NS
L = 16


@functools.partial(jax.jit, static_argnums=(1,))
def _unique_counts_neg(neighbors_flat, n):
    rows_per_w = n // NW
    blk = 8
    nblk = rows_per_w // blk
    chunks = n // L

    mesh = plsc.VectorSubcoreMesh(core_axis_name="c", subcore_axis_name="s")

    @functools.partial(
        pl.kernel,
        out_type=jax.ShapeDtypeStruct((n,), jnp.float32),
        mesh=mesh,
        compiler_params=pltpu.CompilerParams(needs_layout_passes=False),
        scratch_types=[
            pltpu.VMEM((blk * n,), jnp.int32),
            pltpu.VMEM((blk * n,), jnp.int32),
            pltpu.VMEM((n,), jnp.int32),
            pltpu.VMEM((rows_per_w,), jnp.float32),
            pltpu.SemaphoreType.DMA,
            pltpu.SemaphoreType.DMA,
        ],
    )
    def body(nb_hbm, out_hbm, buf0, buf1, mark, res, sem0, sem1):
        cid = lax.axis_index("c")
        sid = lax.axis_index("s")
        wid = sid * NC + cid
        base_row = wid * rows_per_w
        bufs = (buf0, buf1)
        sems = (sem0, sem1)

        lanes = lax.iota(jnp.int32, L)
        mask0 = lanes == 0
        neg1 = jnp.full((L,), -1, jnp.int32)

        def start_block(b, par):
            src = nb_hbm.at[pl.ds((base_row + b * blk) * n, blk * n)]
            pltpu.make_async_copy(src, bufs[par], sems[par]).start()

        def wait_block(par):
            src = nb_hbm.at[pl.ds(0, blk * n)]
            pltpu.make_async_copy(src, bufs[par], sems[par]).wait()

        @plsc.parallel_loop(0, chunks, unroll=8)
        def _(i):
            mark[pl.ds(i * L, L)] = neg1

        start_block(0, 0)
        start_block(1, 1)

        def pair_body(bp, carry):
            for par in range(2):
                b = 2 * bp + par
                wait_block(par)
                rowbuf = bufs[par]

                for r in range(blk):
                    k = b * blk + r
                    kvec = jnp.full((L,), k, jnp.int32)
                    rbase = r * n

                    @plsc.parallel_loop(0, chunks, unroll=8)
                    def _(j):
                        idx = rowbuf[pl.ds(rbase + j * L, L)]
                        plsc.store_scatter(mark, [idx], kvec)

                    @plsc.parallel_loop(
                        0, chunks, unroll=8, carry=jnp.zeros((L,), jnp.int32)
                    )
                    def acc(j, a):
                        m = mark[pl.ds(j * L, L)]
                        return a + (m == kvec).astype(jnp.int32)

                    total = jnp.sum(acc)
                    val = jnp.full((L,), 0.0, jnp.float32) - total.astype(
                        jnp.float32
                    )
                    plsc.store_scatter(
                        res, [jnp.full((L,), k, jnp.int32)], val, mask=mask0
                    )

                @pl.when(b + 2 < nblk)
                def _():
                    start_block(b + 2, par)
            return carry

        lax.fori_loop(0, nblk // 2, pair_body, 0)
        pltpu.sync_copy(res, out_hbm.at[pl.ds(base_row, rows_per_w)])

    return body(neighbors_flat)


def kernel(vision_features, neighbors, gt, num_views):
    n = neighbors.shape[0]
    return _unique_counts_neg(neighbors.reshape(-1), n)


# speedup vs baseline: 1.7909x; 1.7631x over previous
import functools

import jax
import jax.numpy as jnp
from jax import lax
from jax.experimental import pallas as pl
from jax.experimental.pallas import tpu as pltpu
from jax.experimental.pallas import tpu_sc as plsc

NC = 2
NS = 16
NW = NC * NS
L = 16


@jax.jit
def _unique_counts_neg(neighbors):
    n = neighbors.shape[0]
    rows_per_w = n // NW
    blk = 8
    nblk = rows_per_w // blk
    chunks = n // L

    mesh = plsc.VectorSubcoreMesh(core_axis_name="c", subcore_axis_name="s")

    @functools.partial(
        pl.kernel,
        out_type=jax.ShapeDtypeStruct((n,), jnp.float32),
        mesh=mesh,
        compiler_params=pltpu.CompilerParams(needs_layout_passes=False),
        scratch_types=[
            pltpu.VMEM((blk, n), jnp.int32),
            pltpu.VMEM((blk, n), jnp.int32),
            pltpu.VMEM((n,), jnp.int32),
            pltpu.VMEM((rows_per_w,), jnp.float32),
            pltpu.SemaphoreType.DMA,
            pltpu.SemaphoreType.DMA,
        ],
    )
    def body(nb_hbm, out_hbm, buf0, buf1, mark, res, sem0, sem1):
        cid = lax.axis_index("c")
        sid = lax.axis_index("s")
        wid = sid * NC + cid
        base_row = wid * rows_per_w
        bufs = (buf0, buf1)
        sems = (sem0, sem1)

        lanes = lax.iota(jnp.int32, L)
        mask0 = lanes == 0
        neg1 = jnp.full((L,), -1, jnp.int32)

        def start_block(b, par):
            src = nb_hbm.at[pl.ds(base_row + b * blk, blk)]
            pltpu.make_async_copy(src, bufs[par], sems[par]).start()

        def wait_block(par):
            src = nb_hbm.at[pl.ds(0, blk)]
            pltpu.make_async_copy(src, bufs[par], sems[par]).wait()

        @plsc.parallel_loop(0, chunks, unroll=8)
        def _(i):
            mark[pl.ds(i * L, L)] = neg1

        start_block(0, 0)
        start_block(1, 1)

        def pair_body(bp, carry):
            for par in range(2):
                b = 2 * bp + par
                wait_block(par)
                rowbuf = bufs[par]

                for r in range(blk):
                    k = b * blk + r
                    kvec = jnp.full((L,), k, jnp.int32)

                    @plsc.parallel_loop(0, chunks, unroll=8)
                    def _(j):
                        idx = rowbuf[r, pl.ds(j * L, L)]
                        plsc.store_scatter(mark, [idx], kvec)

                    @plsc.parallel_loop(
                        0, chunks, unroll=8, carry=jnp.zeros((L,), jnp.int32)
                    )
                    def acc(j, a):
                        m = mark[pl.ds(j * L, L)]
                        return a + (m == kvec).astype(jnp.int32)

                    total = jnp.sum(acc)
                    val = jnp.full((L,), 0.0, jnp.float32) - total.astype(
                        jnp.float32
                    )
                    plsc.store_scatter(
                        res, [jnp.full((L,), k, jnp.int32)], val, mask=mask0
                    )

                @pl.when(b + 2 < nblk)
                def _():
                    start_block(b + 2, par)
            return carry

        lax.fori_loop(0, nblk // 2, pair_body, 0)
        pltpu.sync_copy(res, out_hbm.at[pl.ds(base_row, rows_per_w)])

    return body(neighbors)


def kernel(vision_features, neighbors, gt, num_views):
    return _unique_counts_neg(neighbors)
